# baseline (speedup 1.0000x reference)
import functools

import jax
import jax.numpy as jnp
from jax import lax
from jax.experimental import pallas as pl
from jax.experimental.pallas import tpu as pltpu
from jax.experimental.pallas import tpu_sc as plsc

L = 200
D = 64
LANES = 16
BT = 128
TPAD = 129
NBUF = 4


def _pe_table():
    position = jnp.arange(L, dtype=jnp.float32)[:, None]
    div_term = jnp.exp(
        jnp.arange(0, D, 2, dtype=jnp.float32) * (-jnp.log(10000.0) / D)
    )
    pe = jnp.zeros((L, D), dtype=jnp.float32)
    pe = pe.at[:, 0::2].set(jnp.sin(position * div_term))
    pe = pe.at[:, 1::2].set(jnp.cos(position * div_term))
    return pe


@functools.partial(jax.jit, static_argnames=("batch", "vocab"))
def _run(xq, table, pe, *, batch, vocab):
    info = plsc.get_sparse_core_info()
    nc, ns = info.num_cores, info.num_subcores
    nw = nc * ns
    nbt = batch // BT
    n_blocks = L * nbt
    blocks_per_w = n_blocks // nw

    mesh = plsc.VectorSubcoreMesh(core_axis_name="c", subcore_axis_name="s")

    @functools.partial(
        pl.kernel,
        out_type=jax.ShapeDtypeStruct((L, D // 8, nbt, 8, BT), jnp.float32),
        mesh=mesh,
        scratch_types=(
            [pltpu.VMEM((blocks_per_w, BT), jnp.int32),
             pltpu.VMEM((L, D), jnp.float32)]
            + [pltpu.VMEM((BT, D), jnp.float32)] * NBUF
            + [pltpu.VMEM((8, 8, TPAD), jnp.float32)] * NBUF
            + [pltpu.SemaphoreType.DMA] * (2 * NBUF)
        ),
        compiler_params=pltpu.CompilerParams(
            use_tc_tiling_on_sc=False, needs_layout_passes=False
        ),
    )
    def k(x_hbm, table_hbm, pe_hbm, out_hbm, idx_all, pe_v, *bufs):
        gbuf = bufs[:NBUF]
        tbuf = bufs[NBUF:2 * NBUF]
        gsem = bufs[2 * NBUF:3 * NBUF]
        wsem = bufs[3 * NBUF:]
        wid = lax.axis_index("s") * nc + lax.axis_index("c")
        blk_base = wid * blocks_per_w
        pltpu.sync_copy(pe_hbm, pe_v)
        pltpu.sync_copy(x_hbm.at[pl.ds(blk_base, blocks_per_w)], idx_all)

        @plsc.parallel_loop(0, blocks_per_w * BT // LANES, unroll=4)
        def _dbl(j):
            r = j // (BT // LANES)
            sl = pl.ds((j % (BT // LANES)) * LANES, LANES)
            idx_all[r, sl] = idx_all[r, sl] * 2

        iota = lax.iota(jnp.int32, LANES)
        trs = [iota // 8 + (g * 2) for g in range(D // LANES)]
        rrs = [iota % 8 for g in range(D // LANES)]

        def start_gather(kk, b):
            pltpu.async_copy(table_hbm.at[idx_all.at[kk]], gbuf[b], gsem[b])

        def wait_gather(kk, b):
            pltpu.make_async_copy(
                table_hbm.at[idx_all.at[kk]], gbuf[b], gsem[b]
            ).wait()

        def lc_of(kk):
            bid = blk_base + kk
            return (bid // (nbt * 8)) * 8 + bid % 8, (bid // 8) % nbt

        def start_write(kk, b):
            l, tc = lc_of(kk)
            pltpu.async_copy(
                tbuf[b].at[pl.ds(0, 8), pl.ds(0, 8), pl.ds(0, BT)],
                out_hbm.at[l, :, tc],
                wsem[b],
            )

        def wait_write(kk, b):
            l, tc = lc_of(kk)
            pltpu.make_async_copy(
                tbuf[b].at[pl.ds(0, 8), pl.ds(0, 8), pl.ds(0, BT)],
                out_hbm.at[l, :, tc],
                wsem[b],
            ).wait()

        for b in range(NBUF):
            start_gather(b, b)

        @pl.loop(0, blocks_per_w, step=NBUF)
        def _blocks(k2):
            for b in range(NBUF):
                kk = k2 + b
                l, _ = lc_of(kk)
                wait_gather(kk, b)

                @pl.when(kk >= NBUF)
                def _():
                    wait_write(kk - NBUF, b)

                pes = [pe_v[l, pl.ds(g * LANES, LANES)]
                       for g in range(D // LANES)]

                @plsc.parallel_loop(0, BT, unroll=4)
                def _tok(c):
                    col = jnp.broadcast_to(c, (LANES,))
                    for g in range(D // LANES):
                        v = gbuf[b][c, pl.ds(g * LANES, LANES)] + pes[g]
                        plsc.store_scatter(tbuf[b], [trs[g], rrs[g], col], v)

                @pl.when(kk + NBUF < blocks_per_w)
                def _():
                    start_gather(kk + NBUF, b)

                start_write(kk, b)

        for b in range(NBUF):
            wait_write(blocks_per_w - NBUF + b, b)

    return k(xq, table, pe)


def kernel(x, table):
    batch, seq = x.shape
    vocab = table.shape[0]
    nbt = batch // BT
    xq = (x.astype(jnp.int32)
          .reshape(nbt, BT, seq // 8, 8)
          .transpose(2, 0, 3, 1)
          .reshape(seq * nbt, BT))
    pe = _pe_table()
    table2 = jnp.concatenate([table, table], axis=1).reshape(2 * vocab, D)
    out5 = _run(xq, table2, pe, batch=batch, vocab=vocab)
    return out5.transpose(2, 4, 0, 1, 3).reshape(batch, seq, D)


# speedup vs baseline: 1.0467x; 1.0467x over previous
import functools

import jax
import jax.numpy as jnp
from jax import lax
from jax.experimental import pallas as pl
from jax.experimental.pallas import tpu as pltpu
from jax.experimental.pallas import tpu_sc as plsc

L = 200
D = 64
LANES = 16
BT = 128
TPAD = 129
NBUF = 4


def _pe_table():
    position = jnp.arange(L, dtype=jnp.float32)[:, None]
    div_term = jnp.exp(
        jnp.arange(0, D, 2, dtype=jnp.float32) * (-jnp.log(10000.0) / D)
    )
    pe = jnp.zeros((L, D), dtype=jnp.float32)
    pe = pe.at[:, 0::2].set(jnp.sin(position * div_term))
    pe = pe.at[:, 1::2].set(jnp.cos(position * div_term))
    return pe


@functools.partial(jax.jit, static_argnames=("batch", "vocab"))
def _run(xq, table, pe, *, batch, vocab):
    info = plsc.get_sparse_core_info()
    nc, ns = info.num_cores, info.num_subcores
    nw = nc * ns
    nbt = batch // BT
    n_blocks = L * nbt
    blocks_per_w = n_blocks // nw

    mesh = plsc.VectorSubcoreMesh(core_axis_name="c", subcore_axis_name="s")

    @functools.partial(
        pl.kernel,
        out_type=jax.ShapeDtypeStruct((L, D // 8, nbt, 8, BT), jnp.float32),
        mesh=mesh,
        scratch_types=(
            [pltpu.VMEM((blocks_per_w, BT), jnp.int32),
             pltpu.VMEM((L, D), jnp.float32)]
            + [pltpu.VMEM((BT, D), jnp.float32)] * NBUF
            + [pltpu.VMEM((8, 8, TPAD), jnp.float32)] * NBUF
            + [pltpu.SemaphoreType.DMA] * (2 * NBUF)
        ),
        compiler_params=pltpu.CompilerParams(
            use_tc_tiling_on_sc=False, needs_layout_passes=False
        ),
    )
    def k(x_hbm, table_hbm, pe_hbm, out_hbm, idx_all, pe_v, *bufs):
        gbuf = bufs[:NBUF]
        tbuf = bufs[NBUF:2 * NBUF]
        gsem = bufs[2 * NBUF:3 * NBUF]
        wsem = bufs[3 * NBUF:]
        wid = lax.axis_index("s") * nc + lax.axis_index("c")
        blk_base = wid * blocks_per_w
        pltpu.sync_copy(pe_hbm, pe_v)
        pltpu.sync_copy(x_hbm.at[pl.ds(blk_base, blocks_per_w)], idx_all)

        iota = lax.iota(jnp.int32, LANES)
        trs = [iota // 8 + (g * 2) for g in range(D // LANES)]
        rrs = [iota % 8 for g in range(D // LANES)]

        def start_gather(kk, b):
            pltpu.async_copy(table_hbm.at[idx_all.at[kk]], gbuf[b], gsem[b])

        def wait_gather(kk, b):
            pltpu.make_async_copy(
                table_hbm.at[idx_all.at[kk]], gbuf[b], gsem[b]
            ).wait()

        def lc_of(kk):
            bid = blk_base + kk
            return (bid // (nbt * 8)) * 8 + bid % 8, (bid // 8) % nbt

        def start_write(kk, b):
            l, tc = lc_of(kk)
            pltpu.async_copy(
                tbuf[b].at[pl.ds(0, 8), pl.ds(0, 8), pl.ds(0, BT)],
                out_hbm.at[l, :, tc],
                wsem[b],
            )

        def wait_write(kk, b):
            l, tc = lc_of(kk)
            pltpu.make_async_copy(
                tbuf[b].at[pl.ds(0, 8), pl.ds(0, 8), pl.ds(0, BT)],
                out_hbm.at[l, :, tc],
                wsem[b],
            ).wait()

        for b in range(NBUF):
            start_gather(b, b)

        @pl.loop(0, blocks_per_w, step=NBUF)
        def _blocks(k2):
            for b in range(NBUF):
                kk = k2 + b
                l, _ = lc_of(kk)
                wait_gather(kk, b)

                @pl.when(kk >= NBUF)
                def _():
                    wait_write(kk - NBUF, b)

                pes = [pe_v[l, pl.ds(g * LANES, LANES)]
                       for g in range(D // LANES)]

                @plsc.parallel_loop(0, BT, unroll=4)
                def _tok(c):
                    col = jnp.broadcast_to(c, (LANES,))
                    for g in range(D // LANES):
                        v = gbuf[b][c, pl.ds(g * LANES, LANES)] + pes[g]
                        plsc.store_scatter(tbuf[b], [trs[g], rrs[g], col], v)

                @pl.when(kk + NBUF < blocks_per_w)
                def _():
                    start_gather(kk + NBUF, b)

                start_write(kk, b)

        for b in range(NBUF):
            wait_write(blocks_per_w - NBUF + b, b)

    return k(xq, table, pe)


def kernel(x, table):
    batch, seq = x.shape
    vocab = table.shape[0]
    nbt = batch // BT
    xq = (x.astype(jnp.int32)
          .reshape(nbt, BT, seq // 8, 8)
          .transpose(2, 0, 3, 1)
          .reshape(seq * nbt, BT))
    pe = _pe_table()
    out5 = _run(xq, table, pe, batch=batch, vocab=vocab)
    return out5.transpose(2, 4, 0, 1, 3).reshape(batch, seq, D)
